# Initial kernel scaffold; baseline (speedup 1.0000x reference)
import jax
import jax.numpy as jnp
from jax.experimental import pallas as pl


def kernel(token_ids, w):
    raise NotImplementedError("write your pallas kernel here")


# speedup vs baseline: 1.1035x; 1.1035x over previous
import functools

import jax
import jax.numpy as jnp
from jax import lax
from jax.experimental import pallas as pl
from jax.experimental.pallas import tpu as pltpu
from jax.experimental.pallas import tpu_sc as plsc

_IDX_PER_GATHER = 128


@functools.cache
def _build(B, V, D):
    info = plsc.get_sparse_core_info()
    nw = info.num_cores * info.num_subcores
    b_per_w = B // nw
    G = 8
    C = G * _IDX_PER_GATHER
    S = b_per_w // C
    rows_per_w = b_per_w // _IDX_PER_GATHER
    assert b_per_w * nw == B and S * C == b_per_w

    mesh = plsc.VectorSubcoreMesh(core_axis_name="c", subcore_axis_name="s")

    @functools.partial(
        pl.kernel,
        mesh=mesh,
        out_type=jax.ShapeDtypeStruct((B, D), jnp.float32),
        compiler_params=pltpu.CompilerParams(use_tc_tiling_on_sc=False),
        scratch_types=[
            pltpu.VMEM((rows_per_w, _IDX_PER_GATHER), jnp.int32),
            pltpu.VMEM((C, D), jnp.float32),
            pltpu.SemaphoreType.DMA,
        ],
    )
    def emb(idx_hbm, table_hbm, out_hbm, idx_v, rows_v, gsem):
        wid = lax.axis_index("s") * info.num_cores + lax.axis_index("c")
        pltpu.sync_copy(idx_hbm.at[pl.ds(wid * rows_per_w, rows_per_w)], idx_v)
        obase = wid * b_per_w

        def chunk(s, carry):
            copies = []
            for j in range(G):
                copies.append(
                    pltpu.async_copy(
                        table_hbm.at[idx_v.at[s * G + j]],
                        rows_v.at[pl.ds(j * _IDX_PER_GATHER, _IDX_PER_GATHER)],
                        gsem,
                    )
                )
            for c in copies:
                c.wait()
            pltpu.sync_copy(rows_v, out_hbm.at[pl.ds(obase + s * C, C)])
            return carry

        lax.fori_loop(0, S, chunk, 0)

    return emb


@jax.jit
def kernel(token_ids, w):
    B = token_ids.size
    idx2d = token_ids.reshape(B // _IDX_PER_GATHER, _IDX_PER_GATHER)
    out = _build(B, w.shape[0], w.shape[1])(idx2d.astype(jnp.int32), w)
    return out.reshape(*token_ids.shape, w.shape[1])


# speedup vs baseline: 1.1135x; 1.0091x over previous
import functools

import jax
import jax.numpy as jnp
from jax import lax
from jax.experimental import pallas as pl
from jax.experimental.pallas import tpu as pltpu
from jax.experimental.pallas import tpu_sc as plsc

_IDX_PER_GATHER = 128


@functools.cache
def _build(B, V, D):
    info = plsc.get_sparse_core_info()
    nw = info.num_cores * info.num_subcores
    b_per_w = B // nw
    G = 5
    C = G * _IDX_PER_GATHER
    NB = 4
    S = b_per_w // C
    P = S // NB
    rows_per_w = b_per_w // _IDX_PER_GATHER
    assert b_per_w * nw == B and S * C == b_per_w and P * NB == S and P >= 2

    mesh = plsc.VectorSubcoreMesh(core_axis_name="c", subcore_axis_name="s")

    @functools.partial(
        pl.kernel,
        mesh=mesh,
        out_type=jax.ShapeDtypeStruct((B, D), jnp.float32),
        compiler_params=pltpu.CompilerParams(use_tc_tiling_on_sc=False),
        scratch_types=[
            pltpu.VMEM((rows_per_w, _IDX_PER_GATHER), jnp.int32),
            pltpu.VMEM((NB, C, D), jnp.float32),
            [pltpu.SemaphoreType.DMA] * NB,
            [pltpu.SemaphoreType.DMA] * NB,
        ],
    )
    def emb(idx_hbm, table_hbm, out_hbm, idx_v, rows_v, gsems, ssems):
        wid = lax.axis_index("s") * info.num_cores + lax.axis_index("c")
        pltpu.sync_copy(idx_hbm.at[pl.ds(wid * rows_per_w, rows_per_w)], idx_v)
        obase = wid * b_per_w

        def fire(c, b):
            for j in range(G):
                pltpu.async_copy(
                    table_hbm.at[idx_v.at[c * G + j]],
                    rows_v.at[b, pl.ds(j * _IDX_PER_GATHER, _IDX_PER_GATHER)],
                    gsems[b],
                )

        def drain_gathers(b):
            for _ in range(G):
                pltpu.make_async_copy(
                    table_hbm.at[idx_v.at[0]], rows_v.at[b, pl.ds(0, _IDX_PER_GATHER)],
                    gsems[b],
                ).wait()

        def wait_store(b):
            pltpu.make_async_copy(
                rows_v.at[b], out_hbm.at[pl.ds(obase, C)], ssems[b]
            ).wait()

        fire(0, 0)
        fire(1, 1)

        def rev(p, carry):
            for b in range(NB):
                s = p * NB + b
                c = s + 2
                bf = (b + 2) % NB
                if b < 2:
                    @pl.when(p >= 1)
                    def _():
                        wait_store(bf)
                    fire(c, bf)
                else:
                    @pl.when(p < P - 1)
                    def _():
                        wait_store(bf)
                        fire(c, bf)
                drain_gathers(b)
                pltpu.async_copy(
                    rows_v.at[b], out_hbm.at[pl.ds(obase + s * C, C)], ssems[b]
                )
            return carry

        lax.fori_loop(0, P, rev, 0)
        for b in range(NB):
            wait_store(b)

    return emb


@jax.jit
def kernel(token_ids, w):
    B = token_ids.size
    idx2d = token_ids.reshape(B // _IDX_PER_GATHER, _IDX_PER_GATHER)
    out = _build(B, w.shape[0], w.shape[1])(idx2d.astype(jnp.int32), w)
    return out.reshape(*token_ids.shape, w.shape[1])


# speedup vs baseline: 1.8098x; 1.6253x over previous
import functools

import jax
import jax.numpy as jnp
from jax import lax
from jax.experimental import pallas as pl
from jax.experimental.pallas import tpu as pltpu
from jax.experimental.pallas import tpu_sc as plsc

_IDX_PER_GATHER = 128


@functools.cache
def _build(B, T, V, D):
    info = plsc.get_sparse_core_info()
    nw = info.num_cores * info.num_subcores
    b_per_w = B // nw
    SEQ_CHUNK = 64
    C = SEQ_CHUNK * T
    G = C // _IDX_PER_GATHER
    S = b_per_w // C
    seq_per_w = b_per_w // T
    rows_per_w = b_per_w // _IDX_PER_GATHER
    assert b_per_w * nw == B and S * C == b_per_w and G * _IDX_PER_GATHER == C

    mesh = plsc.VectorSubcoreMesh(core_axis_name="c", subcore_axis_name="s")

    @functools.partial(
        pl.kernel,
        mesh=mesh,
        out_type=jax.ShapeDtypeStruct((B // T, T, D), jnp.float32),
        compiler_params=pltpu.CompilerParams(use_tc_tiling_on_sc=False),
        scratch_types=[
            pltpu.VMEM((rows_per_w, _IDX_PER_GATHER), jnp.int32),
            pltpu.VMEM((C, D), jnp.float32),
            pltpu.SemaphoreType.DMA,
            pltpu.SemaphoreType.DMA,
        ],
    )
    def emb(idx_hbm, table_hbm, out_hbm, idx_v, rows_v, gsem, ssem):
        wid = lax.axis_index("s") * info.num_cores + lax.axis_index("c")
        pltpu.sync_copy(idx_hbm.at[pl.ds(wid * rows_per_w, rows_per_w)], idx_v)
        sbase = wid * seq_per_w

        def wait_seq_stores(n):
            for _ in range(n):
                pltpu.make_async_copy(
                    rows_v.at[pl.ds(0, T)], out_hbm.at[sbase], ssem
                ).wait()

        def chunk(s, carry):
            @pl.when(s > 0)
            def _():
                wait_seq_stores(SEQ_CHUNK)
            copies = []
            for j in range(G):
                copies.append(
                    pltpu.async_copy(
                        table_hbm.at[idx_v.at[s * G + j]],
                        rows_v.at[pl.ds(j * _IDX_PER_GATHER, _IDX_PER_GATHER)],
                        gsem,
                    )
                )
            q_done = 0
            for j in range(G):
                copies[j].wait()
                q_next = ((j + 1) * _IDX_PER_GATHER) // T
                for q in range(q_done, q_next):
                    pltpu.async_copy(
                        rows_v.at[pl.ds(q * T, T)],
                        out_hbm.at[sbase + s * SEQ_CHUNK + q],
                        ssem,
                    )
                q_done = q_next
            return carry

        lax.fori_loop(0, S, chunk, 0)
        wait_seq_stores(SEQ_CHUNK)

    return emb


@jax.jit
def kernel(token_ids, w):
    B = token_ids.size
    T = token_ids.shape[-1]
    idx2d = token_ids.reshape(B // _IDX_PER_GATHER, _IDX_PER_GATHER)
    return _build(B, T, w.shape[0], w.shape[1])(idx2d.astype(jnp.int32), w)


# speedup vs baseline: 1.8136x; 1.0021x over previous
import functools

import jax
import jax.numpy as jnp
from jax import lax
from jax.experimental import pallas as pl
from jax.experimental.pallas import tpu as pltpu
from jax.experimental.pallas import tpu_sc as plsc

_IDX_PER_GATHER = 128


@functools.cache
def _build(B, T, V, D):
    info = plsc.get_sparse_core_info()
    NC = info.num_cores
    nw = NC * info.num_subcores
    b_per_w = B // nw
    SEQ_CHUNK = 64
    C = SEQ_CHUNK * T
    G = C // _IDX_PER_GATHER
    S = b_per_w // C
    seq_per_w = b_per_w // T
    rows_per_w = b_per_w // _IDX_PER_GATHER
    assert b_per_w * nw == B and S * C == b_per_w and G * _IDX_PER_GATHER == C

    mesh = plsc.VectorSubcoreMesh(core_axis_name="c", subcore_axis_name="s")

    @functools.partial(
        pl.kernel,
        mesh=mesh,
        out_type=jax.ShapeDtypeStruct((B // T, T, D), jnp.float32),
        compiler_params=pltpu.CompilerParams(use_tc_tiling_on_sc=False),
        scratch_types=[
            pltpu.VMEM((rows_per_w, _IDX_PER_GATHER), jnp.int32),
            pltpu.VMEM((C, D), jnp.float32),
            pltpu.SemaphoreType.DMA,
            pltpu.SemaphoreType.DMA,
        ],
    )
    def emb(idx_hbm, table_hbm, out_hbm, idx_v, rows_v, gsem, ssem):
        wid = lax.axis_index("s") * NC + lax.axis_index("c")
        pltpu.sync_copy(idx_hbm.at[pl.ds(wid * rows_per_w, rows_per_w)], idx_v)
        sbase = wid * seq_per_w

        def wait_seq_stores(n):
            for _ in range(n):
                pltpu.make_async_copy(
                    rows_v.at[pl.ds(0, T)], out_hbm.at[sbase], ssem
                ).wait()

        def chunk(s, carry):
            @pl.when(s > 0)
            def _():
                wait_seq_stores(SEQ_CHUNK)
            copies = []
            for j in range(G):
                copies.append(
                    pltpu.async_copy(
                        table_hbm.at[idx_v.at[s * G + j]],
                        rows_v.at[pl.ds(j * _IDX_PER_GATHER, _IDX_PER_GATHER)],
                        gsem,
                    )
                )
            q_done = 0
            for j in range(G):
                copies[j].wait()
                q_next = ((j + 1) * _IDX_PER_GATHER) // T
                for q in range(q_done, q_next):
                    pltpu.async_copy(
                        rows_v.at[pl.ds(q * T, T)],
                        out_hbm.at[sbase + s * SEQ_CHUNK + q],
                        ssem,
                    )
                q_done = q_next
            return carry

        lax.fori_loop(0, S, chunk, 0)
        wait_seq_stores(SEQ_CHUNK)

    return emb


@jax.jit
def kernel(token_ids, w):
    B = token_ids.size
    T = token_ids.shape[-1]
    idx2d = token_ids.reshape(B // _IDX_PER_GATHER, _IDX_PER_GATHER)
    return _build(B, T, w.shape[0], w.shape[1])(idx2d.astype(jnp.int32), w)
